# Initial kernel scaffold; baseline (speedup 1.0000x reference)
import jax
import jax.numpy as jnp
from jax.experimental import pallas as pl


def kernel(latents, codebook):
    raise NotImplementedError("write your pallas kernel here")


# speedup vs baseline: 1.0358x; 1.0358x over previous
import functools

import jax
import jax.numpy as jnp
from jax.experimental import pallas as pl
from jax.experimental.pallas import tpu as pltpu

BETA = 0.25
ROWS = 16384
D = 64
K = 1024
BR = 2048


def _vq_body(flat_ref, cb_ref, q_ref, idx_ref, loss_ref):
    f = flat_ref[...]
    c = cb_ref[...]
    a = jnp.sum(f * f, axis=1, keepdims=True)
    b = jnp.sum(c * c, axis=1)
    mm = jax.lax.dot_general(
        f, c, (((1,), (1,)), ((), ())),
        preferred_element_type=jnp.float32)
    dist = (a + b[None, :]) - 2.0 * mm
    m = jnp.min(dist, axis=1, keepdims=True)
    iota = jax.lax.broadcasted_iota(jnp.int32, dist.shape, 1)
    idx = jnp.min(jnp.where(dist == m, iota, jnp.int32(K)), axis=1,
                  keepdims=True)
    idx_ref[...] = idx
    onehot = (iota == idx).astype(jnp.float32)
    q = jax.lax.dot_general(
        onehot, c, (((1,), (0,)), ((), ())),
        preferred_element_type=jnp.float32,
        precision=jax.lax.Precision.HIGHEST)
    q_ref[...] = q
    diff = q - f
    part = jnp.sum(diff * diff)
    prev = jnp.where(pl.program_id(0) == 0, 0.0, loss_ref[0, 0])
    loss_ref[0, 0] = prev + part


def kernel(latents, codebook):
    B, d, H, W = latents.shape
    lat_t = jnp.transpose(latents, (0, 2, 3, 1))
    flat = lat_t.reshape(-1, d)
    n = B * H * W
    grid = n // BR
    q, idx, loss = pl.pallas_call(
        _vq_body,
        grid=(grid,),
        in_specs=[
            pl.BlockSpec((BR, d), lambda i: (i, 0)),
            pl.BlockSpec((K, d), lambda i: (0, 0)),
        ],
        out_specs=[
            pl.BlockSpec((BR, d), lambda i: (i, 0)),
            pl.BlockSpec((BR, 1), lambda i: (i, 0)),
            pl.BlockSpec(memory_space=pltpu.SMEM, block_shape=(1, 1),
                         index_map=lambda i: (0, 0)),
        ],
        out_shape=[
            jax.ShapeDtypeStruct((n, d), jnp.float32),
            jax.ShapeDtypeStruct((n, 1), jnp.int32),
            jax.ShapeDtypeStruct((1, 1), jnp.float32),
        ],
    )(flat, codebook)
    quantized = jnp.transpose(q.reshape(B, H, W, d), (0, 3, 1, 2))
    vq_loss = (1.0 + BETA) * loss[0, 0] / (n * d)
    return quantized, vq_loss


# speedup vs baseline: 1.9399x; 1.8729x over previous
import functools

import jax
import jax.numpy as jnp
from jax.experimental import pallas as pl
from jax.experimental.pallas import tpu as pltpu

BETA = 0.25
ROWS = 16384
D = 64
K = 1024
BR = 2048


def _vq_body(flat_ref, cb_ref, q_ref, idx_ref, loss_ref):
    f = flat_ref[...]
    c = cb_ref[...]
    a = jnp.sum(f * f, axis=1, keepdims=True)
    b = jnp.sum(c * c, axis=1)
    mm = jax.lax.dot_general(
        f, c, (((1,), (1,)), ((), ())),
        preferred_element_type=jnp.float32)
    dist = (a + b[None, :]) - 2.0 * mm
    m = jnp.min(dist, axis=1, keepdims=True)
    iota = jax.lax.broadcasted_iota(jnp.int32, dist.shape, 1)
    idx = jnp.min(jnp.where(dist == m, iota, jnp.int32(K)), axis=1,
                  keepdims=True)
    idx_ref[...] = idx
    onehot = (iota == idx).astype(jnp.float32)
    q = jax.lax.dot_general(
        onehot, c, (((1,), (0,)), ((), ())),
        preferred_element_type=jnp.float32)
    q_ref[...] = q
    diff = q - f
    part = jnp.sum(diff * diff)
    prev = jnp.where(pl.program_id(0) == 0, 0.0, loss_ref[0, 0])
    loss_ref[0, 0] = prev + part


def kernel(latents, codebook):
    B, d, H, W = latents.shape
    lat_t = jnp.transpose(latents, (0, 2, 3, 1))
    flat = lat_t.reshape(-1, d)
    n = B * H * W
    grid = n // BR
    q, idx, loss = pl.pallas_call(
        _vq_body,
        grid=(grid,),
        in_specs=[
            pl.BlockSpec((BR, d), lambda i: (i, 0)),
            pl.BlockSpec((K, d), lambda i: (0, 0)),
        ],
        out_specs=[
            pl.BlockSpec((BR, d), lambda i: (i, 0)),
            pl.BlockSpec((BR, 1), lambda i: (i, 0)),
            pl.BlockSpec(memory_space=pltpu.SMEM, block_shape=(1, 1),
                         index_map=lambda i: (0, 0)),
        ],
        out_shape=[
            jax.ShapeDtypeStruct((n, d), jnp.float32),
            jax.ShapeDtypeStruct((n, 1), jnp.int32),
            jax.ShapeDtypeStruct((1, 1), jnp.float32),
        ],
    )(flat, codebook)
    quantized = jnp.transpose(q.reshape(B, H, W, d), (0, 3, 1, 2))
    vq_loss = (1.0 + BETA) * loss[0, 0] / (n * d)
    return quantized, vq_loss
